# baseline (speedup 1.0000x reference)
import functools

import jax
import jax.numpy as jnp
from jax import lax
from jax.experimental import pallas as pl
from jax.experimental.pallas import tpu as pltpu
from jax.experimental.pallas import tpu_sc as plsc

N = 10000
E = 320000
D = 128
G = 64

NPAD = 10240
NCORES = 2
NSUB = 16
NW = NCORES * NSUB
CH = 128
EPW = 10240
E_PAD = EPW * NW
NCHUNK = EPW // CH
RPT = NPAD // NSUB

_MESH = plsc.VectorSubcoreMesh(core_axis_name="c", subcore_axis_name="s")


@functools.partial(
    pl.kernel,
    out_type=jax.ShapeDtypeStruct((NCORES, NPAD, D), jnp.float32),
    mesh=_MESH,
    scratch_types=[
        pltpu.VMEM_SHARED((NPAD, D), jnp.float32),
        pltpu.VMEM((CH, D), jnp.float32),
        pltpu.VMEM((CH,), jnp.int32),
    ],
)
def _sc_degree(dst_hbm, z_hbm, o_hbm, out_hbm, shared, ones_v, didx_v):
    c = lax.axis_index("c")
    s = lax.axis_index("s")
    wid = s * NCORES + c

    pltpu.sync_copy(z_hbm.at[pl.ds(s * RPT, RPT)],
                    shared.at[pl.ds(s * RPT, RPT)])
    pltpu.sync_copy(o_hbm, ones_v)

    plsc.subcore_barrier()

    @pl.loop(0, NCHUNK)
    def _chunk(ci):
        base = wid * EPW + ci * CH
        pltpu.sync_copy(dst_hbm.at[pl.ds(base, CH)], didx_v)
        pltpu.sync_copy(ones_v, shared.at[didx_v], add=True)

    plsc.subcore_barrier()
    pltpu.sync_copy(shared.at[pl.ds(s * RPT, RPT)],
                    out_hbm.at[c, pl.ds(s * RPT, RPT)])


@functools.partial(
    pl.kernel,
    out_type=jax.ShapeDtypeStruct((NCORES, NPAD, D), jnp.float32),
    mesh=_MESH,
    scratch_types=[
        pltpu.VMEM_SHARED((NPAD, D), jnp.float32),
        pltpu.VMEM((CH,), jnp.int32),
        pltpu.VMEM((CH,), jnp.int32),
        pltpu.VMEM((CH,), jnp.int32),
        pltpu.VMEM((CH,), jnp.int32),
        pltpu.VMEM((CH, D), jnp.float32),
        pltpu.VMEM((CH, D), jnp.float32),
        pltpu.SemaphoreType.DMA,
        pltpu.SemaphoreType.DMA,
        pltpu.SemaphoreType.DMA,
        pltpu.SemaphoreType.DMA,
    ],
)
def _sc_scatter(table_hbm, src_hbm, dst_hbm, z_hbm, out_hbm,
                shared, sidx0, didx0, sidx1, didx1, rows0, rows1,
                gsem0, gsem1, ssem0, ssem1):
    c = lax.axis_index("c")
    s = lax.axis_index("s")
    wid = s * NCORES + c
    base = wid * EPW

    pltpu.sync_copy(z_hbm.at[pl.ds(s * RPT, RPT)],
                    shared.at[pl.ds(s * RPT, RPT)])

    plsc.subcore_barrier()

    def load_idx(sref, dref, ci):
        pltpu.sync_copy(src_hbm.at[pl.ds(base + ci * CH, CH)], sref)
        pltpu.sync_copy(dst_hbm.at[pl.ds(base + ci * CH, CH)], dref)

    load_idx(sidx0, didx0, 0)
    pltpu.async_copy(table_hbm.at[sidx0], rows0, gsem0)
    load_idx(sidx1, didx1, 1)
    pltpu.async_copy(table_hbm.at[sidx1], rows1, gsem1)

    NP2 = NCHUNK // 2

    @pl.loop(0, NP2)
    def _pair(p):
        pltpu.make_async_copy(table_hbm.at[sidx0], rows0, gsem0).wait()
        pltpu.async_copy(rows0, shared.at[didx0], ssem0, add=True)
        pltpu.make_async_copy(table_hbm.at[sidx1], rows1, gsem1).wait()
        pltpu.async_copy(rows1, shared.at[didx1], ssem1, add=True)
        pltpu.make_async_copy(rows0, shared.at[didx0], ssem0).wait()

        @pl.when(p < NP2 - 1)
        def _pre0():
            load_idx(sidx0, didx0, 2 * p + 2)
            pltpu.async_copy(table_hbm.at[sidx0], rows0, gsem0)

        pltpu.make_async_copy(rows1, shared.at[didx1], ssem1).wait()

        @pl.when(p < NP2 - 1)
        def _pre1():
            load_idx(sidx1, didx1, 2 * p + 3)
            pltpu.async_copy(table_hbm.at[sidx1], rows1, gsem1)

    plsc.subcore_barrier()
    pltpu.sync_copy(shared.at[pl.ds(s * RPT, RPT)],
                    out_hbm.at[c, pl.ds(s * RPT, RPT)])


BR = 256
NBLK = NPAD // BR
BR3 = 400
NBLK3 = N // BR3


def _dinv_block(dacc_ref):
    deg = dacc_ref[0, :, 0:1] + dacc_ref[1, :, 0:1] + 1.0
    return lax.rsqrt(deg)


def _k1_body(dacc_ref, x_ref, w_ref, g_ref):
    dinv = _dinv_block(dacc_ref)
    g_ref[...] = dinv * jnp.dot(x_ref[...], w_ref[...],
                                preferred_element_type=jnp.float32)


def _k2_body(dacc_ref, a_ref, g_ref, b_ref, w_ref, o_ref):
    dinv = _dinv_block(dacc_ref)
    h = jnp.maximum(dinv * (a_ref[0] + a_ref[1] + g_ref[...]) + b_ref[...], 0.0)
    o_ref[...] = dinv * jnp.dot(h, w_ref[...],
                                preferred_element_type=jnp.float32)


def _k3_body(dacc_ref, a_ref, g_ref, b_ref, batch_ref, o_ref, p_acc, c_acc):
    i = pl.program_id(0)

    @pl.when(i == 0)
    def _init():
        p_acc[...] = jnp.zeros((G, D), jnp.float32)
        c_acc[...] = jnp.zeros((G, D), jnp.float32)

    dinv = _dinv_block(dacc_ref)
    h = jnp.maximum(dinv * (a_ref[0] + a_ref[1] + g_ref[...]) + b_ref[...], 0.0)
    bvec = batch_ref[0, 0, :]
    gids = lax.broadcasted_iota(jnp.int32, (BR3, G), 1)
    msk = (bvec[:, None] == gids).astype(jnp.float32)
    dn = (((0,), (0,)), ((), ()))
    p_acc[...] += lax.dot_general(msk, h, dn,
                                  preferred_element_type=jnp.float32)
    c_acc[...] += lax.dot_general(msk, jnp.ones((BR3, D), jnp.float32), dn,
                                  preferred_element_type=jnp.float32)

    @pl.when(i == NBLK3 - 1)
    def _fin():
        o_ref[...] = p_acc[...] / jnp.maximum(c_acc[...], 1.0)


_k1 = pl.pallas_call(
    _k1_body,
    grid=(NBLK,),
    in_specs=[
        pl.BlockSpec((NCORES, BR, D), lambda i: (0, i, 0)),
        pl.BlockSpec((BR, D), lambda i: (i, 0)),
        pl.BlockSpec((D, D), lambda i: (0, 0)),
    ],
    out_specs=pl.BlockSpec((BR, D), lambda i: (i, 0)),
    out_shape=jax.ShapeDtypeStruct((NPAD, D), jnp.float32),
)

_k2 = pl.pallas_call(
    _k2_body,
    grid=(NBLK,),
    in_specs=[
        pl.BlockSpec((NCORES, BR, D), lambda i: (0, i, 0)),
        pl.BlockSpec((NCORES, BR, D), lambda i: (0, i, 0)),
        pl.BlockSpec((BR, D), lambda i: (i, 0)),
        pl.BlockSpec((1, D), lambda i: (0, 0)),
        pl.BlockSpec((D, D), lambda i: (0, 0)),
    ],
    out_specs=pl.BlockSpec((BR, D), lambda i: (i, 0)),
    out_shape=jax.ShapeDtypeStruct((NPAD, D), jnp.float32),
)

_k3 = pl.pallas_call(
    _k3_body,
    grid=(NBLK3,),
    in_specs=[
        pl.BlockSpec((NCORES, BR3, D), lambda i: (0, i, 0)),
        pl.BlockSpec((NCORES, BR3, D), lambda i: (0, i, 0)),
        pl.BlockSpec((BR3, D), lambda i: (i, 0)),
        pl.BlockSpec((1, D), lambda i: (0, 0)),
        pl.BlockSpec((1, 1, BR3), lambda i: (i, 0, 0)),
    ],
    out_specs=pl.BlockSpec((G, D), lambda i: (0, 0)),
    out_shape=jax.ShapeDtypeStruct((G, D), jnp.float32),
    scratch_shapes=[
        pltpu.VMEM((G, D), jnp.float32),
        pltpu.VMEM((G, D), jnp.float32),
    ],
)


def kernel(x, edge_index, batch, W1, b1, W2, b2):
    pad = E_PAD - E
    src = jnp.concatenate([edge_index[0], jnp.zeros((pad,), jnp.int32)])
    dummy = N + (jnp.arange(pad, dtype=jnp.int32) % (NPAD - N))
    dst = jnp.concatenate([edge_index[1], dummy])
    x_pad = jnp.pad(x, ((0, NPAD - N), (0, 0)))
    b1r = b1.reshape(1, D)
    b2r = b2.reshape(1, D)
    batch_r = batch.reshape(NBLK3, 1, BR3)

    oD = jnp.ones((CH, D), jnp.float32)
    zD = jnp.zeros((NPAD, D), jnp.float32)

    dacc = _sc_degree(dst, zD, oD)
    g1 = _k1(dacc, x_pad, W1)
    a1 = _sc_scatter(g1, src, dst, zD)
    g2 = _k2(dacc, a1, g1, b1r, W2)
    a2 = _sc_scatter(g2, src, dst, zD)
    return _k3(dacc, a2, g2, b2r, batch_r)


# speedup vs baseline: 1.0542x; 1.0542x over previous
import functools

import jax
import jax.numpy as jnp
from jax import lax
from jax.experimental import pallas as pl
from jax.experimental.pallas import tpu as pltpu
from jax.experimental.pallas import tpu_sc as plsc

N = 10000
E = 320000
D = 128
G = 64

NPAD = 10240
NCORES = 2
NSUB = 16
NW = NCORES * NSUB
CH = 128
EPW = 10240
E_PAD = EPW * NW
NCHUNK = EPW // CH
RPT = NPAD // NSUB

_MESH = plsc.VectorSubcoreMesh(core_axis_name="c", subcore_axis_name="s")


@functools.partial(
    pl.kernel,
    out_type=jax.ShapeDtypeStruct((NCORES, NPAD, D), jnp.float32),
    mesh=_MESH,
    scratch_types=[
        pltpu.VMEM_SHARED((NPAD, D), jnp.float32),
        pltpu.VMEM((CH, D), jnp.float32),
        pltpu.VMEM((CH,), jnp.int32),
    ],
)
def _sc_degree(dst_hbm, z_hbm, o_hbm, out_hbm, shared, ones_v, didx_v):
    c = lax.axis_index("c")
    s = lax.axis_index("s")
    wid = s * NCORES + c

    pltpu.sync_copy(z_hbm.at[pl.ds(s * RPT, RPT)],
                    shared.at[pl.ds(s * RPT, RPT)])
    pltpu.sync_copy(o_hbm, ones_v)

    plsc.subcore_barrier()

    @pl.loop(0, NCHUNK)
    def _chunk(ci):
        base = wid * EPW + ci * CH
        pltpu.sync_copy(dst_hbm.at[pl.ds(base, CH)], didx_v)
        pltpu.sync_copy(ones_v, shared.at[didx_v], add=True)

    plsc.subcore_barrier()
    pltpu.sync_copy(shared.at[pl.ds(s * RPT, RPT)],
                    out_hbm.at[c, pl.ds(s * RPT, RPT)])


@functools.partial(
    pl.kernel,
    out_type=jax.ShapeDtypeStruct((NCORES, NPAD, D), jnp.float32),
    mesh=_MESH,
    scratch_types=[
        pltpu.VMEM_SHARED((NPAD, D), jnp.float32),
        pltpu.VMEM((CH,), jnp.int32),
        pltpu.VMEM((CH,), jnp.int32),
        pltpu.VMEM((CH,), jnp.int32),
        pltpu.VMEM((CH,), jnp.int32),
        pltpu.VMEM((CH, D), jnp.float32),
        pltpu.VMEM((CH, D), jnp.float32),
        pltpu.SemaphoreType.DMA,
        pltpu.SemaphoreType.DMA,
    ],
)
def _sc_scatter(table_hbm, src_hbm, dst_hbm, z_hbm, out_hbm,
                shared, sidx0, didx0, sidx1, didx1, rows0, rows1,
                gsem0, gsem1):
    c = lax.axis_index("c")
    s = lax.axis_index("s")
    wid = s * NCORES + c
    base = wid * EPW

    pltpu.sync_copy(z_hbm.at[pl.ds(s * RPT, RPT)],
                    shared.at[pl.ds(s * RPT, RPT)])

    plsc.subcore_barrier()

    def load_idx(sref, dref, ci):
        pltpu.sync_copy(src_hbm.at[pl.ds(base + ci * CH, CH)], sref)
        pltpu.sync_copy(dst_hbm.at[pl.ds(base + ci * CH, CH)], dref)

    load_idx(sidx0, didx0, 0)
    pltpu.async_copy(table_hbm.at[sidx0], rows0, gsem0)

    NP2 = NCHUNK // 2

    @pl.loop(0, NP2)
    def _pair(p):
        load_idx(sidx1, didx1, 2 * p + 1)
        pltpu.async_copy(table_hbm.at[sidx1], rows1, gsem1)
        pltpu.make_async_copy(table_hbm.at[sidx0], rows0, gsem0).wait()
        pltpu.sync_copy(rows0, shared.at[didx0], add=True)

        @pl.when(p < NP2 - 1)
        def _pre0():
            load_idx(sidx0, didx0, 2 * p + 2)
            pltpu.async_copy(table_hbm.at[sidx0], rows0, gsem0)

        pltpu.make_async_copy(table_hbm.at[sidx1], rows1, gsem1).wait()
        pltpu.sync_copy(rows1, shared.at[didx1], add=True)

    plsc.subcore_barrier()
    pltpu.sync_copy(shared.at[pl.ds(s * RPT, RPT)],
                    out_hbm.at[c, pl.ds(s * RPT, RPT)])


BR = 256
NBLK = NPAD // BR
BR3 = 400
NBLK3 = N // BR3


def _dinv_block(dacc_ref):
    deg = dacc_ref[0, :, 0:1] + dacc_ref[1, :, 0:1] + 1.0
    return lax.rsqrt(deg)


def _k1_body(dacc_ref, x_ref, w_ref, g_ref):
    dinv = _dinv_block(dacc_ref)
    g_ref[...] = dinv * jnp.dot(x_ref[...], w_ref[...],
                                preferred_element_type=jnp.float32)


def _k2_body(dacc_ref, a_ref, g_ref, b_ref, w_ref, o_ref):
    dinv = _dinv_block(dacc_ref)
    h = jnp.maximum(dinv * (a_ref[0] + a_ref[1] + g_ref[...]) + b_ref[...], 0.0)
    o_ref[...] = dinv * jnp.dot(h, w_ref[...],
                                preferred_element_type=jnp.float32)


def _k3_body(dacc_ref, a_ref, g_ref, b_ref, batch_ref, o_ref, p_acc, c_acc):
    i = pl.program_id(0)

    @pl.when(i == 0)
    def _init():
        p_acc[...] = jnp.zeros((G, D), jnp.float32)
        c_acc[...] = jnp.zeros((G, D), jnp.float32)

    dinv = _dinv_block(dacc_ref)
    h = jnp.maximum(dinv * (a_ref[0] + a_ref[1] + g_ref[...]) + b_ref[...], 0.0)
    bvec = batch_ref[0, 0, :]
    gids = lax.broadcasted_iota(jnp.int32, (BR3, G), 1)
    msk = (bvec[:, None] == gids).astype(jnp.float32)
    dn = (((0,), (0,)), ((), ()))
    p_acc[...] += lax.dot_general(msk, h, dn,
                                  preferred_element_type=jnp.float32)
    c_acc[...] += lax.dot_general(msk, jnp.ones((BR3, D), jnp.float32), dn,
                                  preferred_element_type=jnp.float32)

    @pl.when(i == NBLK3 - 1)
    def _fin():
        o_ref[...] = p_acc[...] / jnp.maximum(c_acc[...], 1.0)


_k1 = pl.pallas_call(
    _k1_body,
    grid=(NBLK,),
    in_specs=[
        pl.BlockSpec((NCORES, BR, D), lambda i: (0, i, 0)),
        pl.BlockSpec((BR, D), lambda i: (i, 0)),
        pl.BlockSpec((D, D), lambda i: (0, 0)),
    ],
    out_specs=pl.BlockSpec((BR, D), lambda i: (i, 0)),
    out_shape=jax.ShapeDtypeStruct((NPAD, D), jnp.float32),
)

_k2 = pl.pallas_call(
    _k2_body,
    grid=(NBLK,),
    in_specs=[
        pl.BlockSpec((NCORES, BR, D), lambda i: (0, i, 0)),
        pl.BlockSpec((NCORES, BR, D), lambda i: (0, i, 0)),
        pl.BlockSpec((BR, D), lambda i: (i, 0)),
        pl.BlockSpec((1, D), lambda i: (0, 0)),
        pl.BlockSpec((D, D), lambda i: (0, 0)),
    ],
    out_specs=pl.BlockSpec((BR, D), lambda i: (i, 0)),
    out_shape=jax.ShapeDtypeStruct((NPAD, D), jnp.float32),
)

_k3 = pl.pallas_call(
    _k3_body,
    grid=(NBLK3,),
    in_specs=[
        pl.BlockSpec((NCORES, BR3, D), lambda i: (0, i, 0)),
        pl.BlockSpec((NCORES, BR3, D), lambda i: (0, i, 0)),
        pl.BlockSpec((BR3, D), lambda i: (i, 0)),
        pl.BlockSpec((1, D), lambda i: (0, 0)),
        pl.BlockSpec((1, 1, BR3), lambda i: (i, 0, 0)),
    ],
    out_specs=pl.BlockSpec((G, D), lambda i: (0, 0)),
    out_shape=jax.ShapeDtypeStruct((G, D), jnp.float32),
    scratch_shapes=[
        pltpu.VMEM((G, D), jnp.float32),
        pltpu.VMEM((G, D), jnp.float32),
    ],
)


def kernel(x, edge_index, batch, W1, b1, W2, b2):
    pad = E_PAD - E
    src = jnp.concatenate([edge_index[0], jnp.zeros((pad,), jnp.int32)])
    dummy = N + (jnp.arange(pad, dtype=jnp.int32) % (NPAD - N))
    dst = jnp.concatenate([edge_index[1], dummy])
    x_pad = jnp.pad(x, ((0, NPAD - N), (0, 0)))
    b1r = b1.reshape(1, D)
    b2r = b2.reshape(1, D)
    batch_r = batch.reshape(NBLK3, 1, BR3)

    oD = jnp.ones((CH, D), jnp.float32)
    zD = jnp.zeros((NPAD, D), jnp.float32)

    dacc = _sc_degree(dst, zD, oD)
    g1 = _k1(dacc, x_pad, W1)
    a1 = _sc_scatter(g1, src, dst, zD)
    g2 = _k2(dacc, a1, g1, b1r, W2)
    a2 = _sc_scatter(g2, src, dst, zD)
    return _k3(dacc, a2, g2, b2r, batch_r)


# speedup vs baseline: 2.2281x; 2.1134x over previous
import functools

import jax
import jax.numpy as jnp
from jax import lax
from jax.experimental import pallas as pl
from jax.experimental.pallas import tpu as pltpu
from jax.experimental.pallas import tpu_sc as plsc

N = 10000
E = 320000
D = 128
G = 64

NPAD = 10240
NCORES = 2
NSUB = 16
NW = NCORES * NSUB
CH = 128
EPW = 10240
E_PAD = EPW * NW
NCHUNK = EPW // CH
RPT = NPAD // NSUB

_MESH = plsc.VectorSubcoreMesh(core_axis_name="c", subcore_axis_name="s")


@functools.partial(
    pl.kernel,
    out_type=jax.ShapeDtypeStruct((NCORES, NPAD, D), jnp.float32),
    mesh=_MESH,
    scratch_types=[
        pltpu.VMEM_SHARED((NPAD, D), jnp.float32),
        pltpu.VMEM((CH, D), jnp.float32),
        pltpu.VMEM((CH,), jnp.int32),
    ],
)
def _sc_degree(dst_hbm, z_hbm, o_hbm, out_hbm, shared, ones_v, didx_v):
    c = lax.axis_index("c")
    s = lax.axis_index("s")
    wid = s * NCORES + c

    pltpu.sync_copy(z_hbm.at[pl.ds(s * RPT, RPT)],
                    shared.at[pl.ds(s * RPT, RPT)])
    pltpu.sync_copy(o_hbm, ones_v)

    plsc.subcore_barrier()

    @pl.loop(0, NCHUNK)
    def _chunk(ci):
        base = wid * EPW + ci * CH
        pltpu.sync_copy(dst_hbm.at[pl.ds(base, CH)], didx_v)
        pltpu.sync_copy(ones_v, shared.at[didx_v], add=True)

    plsc.subcore_barrier()
    pltpu.sync_copy(shared.at[pl.ds(s * RPT, RPT)],
                    out_hbm.at[c, pl.ds(s * RPT, RPT)])


@functools.partial(
    pl.kernel,
    out_type=jax.ShapeDtypeStruct((NCORES, NPAD, D), jnp.float32),
    mesh=_MESH,
    scratch_types=[
        pltpu.VMEM_SHARED((NPAD, D), jnp.float32),
        pltpu.VMEM((CH,), jnp.int32),
        pltpu.VMEM((CH,), jnp.int32),
        pltpu.VMEM((CH,), jnp.int32),
        pltpu.VMEM((CH,), jnp.int32),
        pltpu.VMEM((CH, D), jnp.float32),
        pltpu.VMEM((CH, D), jnp.float32),
        pltpu.SemaphoreType.DMA,
        pltpu.SemaphoreType.DMA,
    ],
)
def _sc_scatter(table_hbm, src_hbm, dst_hbm, z_hbm, out_hbm,
                shared, sidx0, didx0, sidx1, didx1, rows0, rows1,
                gsem0, gsem1):
    c = lax.axis_index("c")
    s = lax.axis_index("s")
    wid = s * NCORES + c
    base = wid * EPW

    pltpu.sync_copy(z_hbm.at[pl.ds(s * RPT, RPT)],
                    shared.at[pl.ds(s * RPT, RPT)])

    plsc.subcore_barrier()

    def load_idx(sref, dref, ci):
        pltpu.sync_copy(src_hbm.at[pl.ds(base + ci * CH, CH)], sref)
        pltpu.sync_copy(dst_hbm.at[pl.ds(base + ci * CH, CH)], dref)

    load_idx(sidx0, didx0, 0)
    pltpu.async_copy(table_hbm.at[sidx0], rows0, gsem0)

    NP2 = NCHUNK // 2

    @pl.loop(0, NP2)
    def _pair(p):
        load_idx(sidx1, didx1, 2 * p + 1)
        pltpu.async_copy(table_hbm.at[sidx1], rows1, gsem1)
        pltpu.make_async_copy(table_hbm.at[sidx0], rows0, gsem0).wait()
        pltpu.sync_copy(rows0, shared.at[didx0], add=True)

        @pl.when(p < NP2 - 1)
        def _pre0():
            load_idx(sidx0, didx0, 2 * p + 2)
            pltpu.async_copy(table_hbm.at[sidx0], rows0, gsem0)

        pltpu.make_async_copy(table_hbm.at[sidx1], rows1, gsem1).wait()
        pltpu.sync_copy(rows1, shared.at[didx1], add=True)

    plsc.subcore_barrier()
    pltpu.sync_copy(shared.at[pl.ds(s * RPT, RPT)],
                    out_hbm.at[c, pl.ds(s * RPT, RPT)])


BR = 256
NBLK = NPAD // BR
BR3 = 400
NBLK3 = N // BR3


def _dinv_block(dacc_ref):
    deg = dacc_ref[0, :, 0:1] + dacc_ref[1, :, 0:1] + 1.0
    return lax.rsqrt(deg)


def _k1_body(dacc_ref, x_ref, w_ref, g_ref):
    dinv = _dinv_block(dacc_ref)
    g_ref[...] = dinv * jnp.dot(x_ref[...], w_ref[...],
                                preferred_element_type=jnp.float32)


def _k2_body(dacc_ref, a_ref, g_ref, b_ref, w_ref, o_ref):
    dinv = _dinv_block(dacc_ref)
    h = jnp.maximum(dinv * (a_ref[0] + a_ref[1] + g_ref[...]) + b_ref[...], 0.0)
    o_ref[...] = dinv * jnp.dot(h, w_ref[...],
                                preferred_element_type=jnp.float32)


def _k3_body(dacc_ref, a_ref, g_ref, b_ref, batch_ref, o_ref, p_acc, c_acc):
    i = pl.program_id(0)

    @pl.when(i == 0)
    def _init():
        p_acc[...] = jnp.zeros((G, D), jnp.float32)
        c_acc[...] = jnp.zeros((G, D), jnp.float32)

    dinv = _dinv_block(dacc_ref)
    h = jnp.maximum(dinv * (a_ref[0] + a_ref[1] + g_ref[...]) + b_ref[...], 0.0)
    bvec = batch_ref[0, 0, :]
    gids = lax.broadcasted_iota(jnp.int32, (BR3, G), 1)
    msk = (bvec[:, None] == gids).astype(jnp.float32)
    dn = (((0,), (0,)), ((), ()))
    p_acc[...] += lax.dot_general(msk, h, dn,
                                  preferred_element_type=jnp.float32)
    c_acc[...] += lax.dot_general(msk, jnp.ones((BR3, D), jnp.float32), dn,
                                  preferred_element_type=jnp.float32)

    @pl.when(i == NBLK3 - 1)
    def _fin():
        o_ref[...] = p_acc[...] / jnp.maximum(c_acc[...], 1.0)


_k1 = pl.pallas_call(
    _k1_body,
    grid=(NBLK,),
    in_specs=[
        pl.BlockSpec((NCORES, BR, D), lambda i: (0, i, 0)),
        pl.BlockSpec((BR, D), lambda i: (i, 0)),
        pl.BlockSpec((D, D), lambda i: (0, 0)),
    ],
    out_specs=pl.BlockSpec((BR, D), lambda i: (i, 0)),
    out_shape=jax.ShapeDtypeStruct((NPAD, D), jnp.float32),
)

_k2 = pl.pallas_call(
    _k2_body,
    grid=(NBLK,),
    in_specs=[
        pl.BlockSpec((NCORES, BR, D), lambda i: (0, i, 0)),
        pl.BlockSpec((NCORES, BR, D), lambda i: (0, i, 0)),
        pl.BlockSpec((BR, D), lambda i: (i, 0)),
        pl.BlockSpec((1, D), lambda i: (0, 0)),
        pl.BlockSpec((D, D), lambda i: (0, 0)),
    ],
    out_specs=pl.BlockSpec((BR, D), lambda i: (i, 0)),
    out_shape=jax.ShapeDtypeStruct((NPAD, D), jnp.float32),
)

_k3 = pl.pallas_call(
    _k3_body,
    grid=(NBLK3,),
    in_specs=[
        pl.BlockSpec((NCORES, BR3, D), lambda i: (0, i, 0)),
        pl.BlockSpec((NCORES, BR3, D), lambda i: (0, i, 0)),
        pl.BlockSpec((BR3, D), lambda i: (i, 0)),
        pl.BlockSpec((1, D), lambda i: (0, 0)),
        pl.BlockSpec((1, 1, BR3), lambda i: (i, 0, 0)),
    ],
    out_specs=pl.BlockSpec((G, D), lambda i: (0, 0)),
    out_shape=jax.ShapeDtypeStruct((G, D), jnp.float32),
    scratch_shapes=[
        pltpu.VMEM((G, D), jnp.float32),
        pltpu.VMEM((G, D), jnp.float32),
    ],
)


def kernel(x, edge_index, batch, W1, b1, W2, b2):
    pad = E_PAD - E
    ramp = jnp.arange(pad, dtype=jnp.int32)
    src = jnp.concatenate([edge_index[0], ramp % N])
    dst = jnp.concatenate([edge_index[1], N + ramp % (NPAD - N)])
    x_pad = jnp.pad(x, ((0, NPAD - N), (0, 0)))
    b1r = b1.reshape(1, D)
    b2r = b2.reshape(1, D)
    batch_r = batch.reshape(NBLK3, 1, BR3)

    oD = jnp.ones((CH, D), jnp.float32)
    zD = jnp.zeros((NPAD, D), jnp.float32)

    dacc = _sc_degree(dst, zD, oD)
    g1 = _k1(dacc, x_pad, W1)
    a1 = _sc_scatter(g1, src, dst, zD)
    g2 = _k2(dacc, a1, g1, b1r, W2)
    a2 = _sc_scatter(g2, src, dst, zD)
    return _k3(dacc, a2, g2, b2r, batch_r)
